# baseline (speedup 1.0000x reference)
import functools

import jax
import jax.numpy as jnp
from jax import lax
from jax.experimental import pallas as pl
from jax.experimental.pallas import tpu as pltpu
from jax.experimental.pallas import tpu_sc as plsc

E = 160000
N = 10000
D = 256
W = D // 2
L = 16
NC, NS = 2, 16
NW = NC * NS
EP = 163840
PER_W = EP // NW
B = 64
NBLK = PER_W // B
NRS = 2
NIS = 4
DC = W // L
NP = 10240
ROWS_PER_TILE = NP // NS
HIMASK = -65536

_mesh = plsc.VectorSubcoreMesh(core_axis_name="c", subcore_axis_name="s")

_GATHER_DN = lax.GatherDimensionNumbers(
    offset_dims=(), collapsed_slice_dims=(0,), start_index_map=(0,))


def _rotate(v, perm):
    return lax.gather(v, perm[:, None], _GATHER_DN, slice_sizes=(1,),
                      mode=lax.GatherScatterMode.PROMISE_IN_BOUNDS)


@functools.partial(
    pl.kernel,
    mesh=_mesh,
    out_type=jax.ShapeDtypeStruct((EP // B, B), jnp.float32),
    compiler_params=pltpu.CompilerParams(needs_layout_passes=False),
    scratch_types=[
        pltpu.VMEM_SHARED((NP, W), jnp.int32),
        pltpu.VMEM((NIS, B), jnp.int32),
        pltpu.VMEM((NIS, B), jnp.int32),
        pltpu.VMEM((NRS, B, W), jnp.int32),
        pltpu.VMEM((NRS, B, W), jnp.int32),
        pltpu.VMEM((NRS, B), jnp.float32),
        pltpu.SemaphoreType.DMA,
        pltpu.SemaphoreType.DMA,
        pltpu.SemaphoreType.DMA,
        pltpu.SemaphoreType.DMA,
        pltpu.SemaphoreType.DMA,
        pltpu.SemaphoreType.DMA,
    ],
)
def _decode(z_hbm, sidx_hbm, didx_hbm, out_hbm,
            ztab_v, sidxb, didxb, srows_v, drows_v, outb_v, *sems):
    rsems = sems[:NRS]
    isems = sems[NRS:]
    cid = lax.axis_index("c")
    sid = lax.axis_index("s")
    wid = sid * NC + cid
    wblk = wid * NBLK

    row0 = sid * ROWS_PER_TILE
    pltpu.sync_copy(z_hbm.at[pl.ds(row0, ROWS_PER_TILE)],
                    ztab_v.at[pl.ds(row0, ROWS_PER_TILE)])
    plsc.subcore_barrier()

    lanes = lax.broadcasted_iota(jnp.int32, (L,), 0)
    rots = [(lanes + r) % L for r in (8, 4, 2, 1)]

    def issue_idx(blk, s):
        pltpu.async_copy(sidx_hbm.at[wblk + blk], sidxb.at[s], isems[s])
        pltpu.async_copy(didx_hbm.at[wblk + blk], didxb.at[s], isems[s])

    def wait_idx(s):
        pltpu.make_async_copy(sidx_hbm.at[0], sidxb.at[s], isems[s]).wait()
        pltpu.make_async_copy(didx_hbm.at[0], didxb.at[s], isems[s]).wait()

    def issue_rows(isl, rs):
        pltpu.async_copy(ztab_v.at[sidxb.at[isl]], srows_v.at[rs], rsems[rs])
        pltpu.async_copy(ztab_v.at[didxb.at[isl]], drows_v.at[rs], rsems[rs])

    def wait_rows(isl, rs):
        pltpu.make_async_copy(
            ztab_v.at[sidxb.at[isl]], srows_v.at[rs], rsems[rs]).wait()
        pltpu.make_async_copy(
            ztab_v.at[didxb.at[isl]], drows_v.at[rs], rsems[rs]).wait()

    issue_idx(0, 0)
    issue_idx(1, 1)
    wait_idx(0)
    issue_rows(0, 0)

    def body(g, c):
        for s in range(NIS):
            blk = NIS * g + s
            rs = s % NRS
            sr = srows_v.at[rs]
            dr = drows_v.at[rs]
            ob = outb_v.at[rs]

            @pl.when(blk + 1 < NBLK)
            def _():
                wait_idx((s + 1) % NIS)
                issue_rows((s + 1) % NIS, (rs + 1) % NRS)

            @pl.when(blk + 2 < NBLK)
            def _():
                issue_idx(blk + 2, (s + 2) % NIS)

            wait_rows(s, rs)

            def grp_body(g2, c2):
                gbase = g2 * L

                def edge_body(i, res):
                    e = gbase + i
                    acc = jnp.zeros((L,), jnp.float32)
                    for j in range(DC):
                        sw = sr[e, pl.ds(j * L, L)]
                        dw = dr[e, pl.ds(j * L, L)]
                        sa = plsc.bitcast(sw << 16, jnp.float32)
                        sb = plsc.bitcast(sw & HIMASK, jnp.float32)
                        da = plsc.bitcast(dw << 16, jnp.float32)
                        db = plsc.bitcast(dw & HIMASK, jnp.float32)
                        acc = acc + sa * da + sb * db
                    for perm in rots:
                        acc = acc + _rotate(acc, perm)
                    return lax.select(lanes == i, acc, res)

                res = lax.fori_loop(0, L, edge_body,
                                    jnp.zeros((L,), jnp.float32), unroll=2)
                res = 1.0 / (1.0 + jnp.exp(-res))
                ob[pl.ds(pl.multiple_of(gbase, L), L)] = res
                return c2

            lax.fori_loop(0, B // L, grp_body, 0)

            pltpu.sync_copy(ob, out_hbm.at[wblk + blk])
        return c

    lax.fori_loop(0, NBLK // NIS, body, 0)


def _pack_bf16_words(z):
    zu = lax.bitcast_convert_type(z, jnp.uint32)
    even = zu[:, 0::2]
    odd = zu[:, 1::2]

    def rne(x):
        return (x + jnp.uint32(0x7FFF) + ((x >> 16) & 1)) >> 16

    packed = rne(even) | (rne(odd) << 16)
    words = lax.bitcast_convert_type(packed, jnp.int32)
    return jnp.pad(words, ((0, NP - N), (0, 0)))


def kernel(z, edge_idx):
    idx = edge_idx.astype(jnp.int32)
    pad = EP - E
    sidx = jnp.pad(idx[0], (0, pad)).reshape(EP // B, B)
    didx = jnp.pad(idx[1], (0, pad)).reshape(EP // B, B)
    out = _decode(_pack_bf16_words(z), sidx, didx)
    return out.reshape(-1)[:E]


# speedup vs baseline: 3.6310x; 3.6310x over previous
import functools

import jax
import jax.numpy as jnp
from jax import lax
from jax.experimental import pallas as pl
from jax.experimental.pallas import tpu as pltpu
from jax.experimental.pallas import tpu_sc as plsc

E = 160000
N = 10000
D = 256
W = D // 2
L = 16
NC, NS = 2, 16
NW = NC * NS
EP = 163840
PER_W = EP // NW
B = 64
NBLK = PER_W // B
NRS = 2
NIS = 4
DC = W // L
NP = 10240
ROWS_PER_TILE = NP // NS
HIMASK = -65536

_mesh = plsc.VectorSubcoreMesh(core_axis_name="c", subcore_axis_name="s")

_GATHER_DN = lax.GatherDimensionNumbers(
    offset_dims=(), collapsed_slice_dims=(0,), start_index_map=(0,))


def _rotate(v, perm):
    return lax.gather(v, perm[:, None], _GATHER_DN, slice_sizes=(1,),
                      mode=lax.GatherScatterMode.PROMISE_IN_BOUNDS)


@functools.partial(
    pl.kernel,
    mesh=_mesh,
    out_type=jax.ShapeDtypeStruct((EP // B, B), jnp.float32),
    compiler_params=pltpu.CompilerParams(needs_layout_passes=False),
    scratch_types=[
        pltpu.VMEM_SHARED((NP, W), jnp.int32),
        pltpu.VMEM((NIS, B), jnp.int32),
        pltpu.VMEM((NIS, B), jnp.int32),
        pltpu.VMEM((NRS, B, W), jnp.int32),
        pltpu.VMEM((NRS, B, W), jnp.int32),
        pltpu.VMEM((NRS, B), jnp.float32),
        pltpu.SemaphoreType.DMA,
        pltpu.SemaphoreType.DMA,
        pltpu.SemaphoreType.DMA,
        pltpu.SemaphoreType.DMA,
        pltpu.SemaphoreType.DMA,
        pltpu.SemaphoreType.DMA,
    ],
)
def _decode(z_hbm, sidx_hbm, didx_hbm, out_hbm,
            ztab_v, sidxb, didxb, srows_v, drows_v, outb_v, *sems):
    rsems = sems[:NRS]
    isems = sems[NRS:]
    cid = lax.axis_index("c")
    sid = lax.axis_index("s")
    wid = sid * NC + cid
    wblk = wid * NBLK

    row0 = sid * ROWS_PER_TILE
    pltpu.sync_copy(z_hbm.at[pl.ds(row0, ROWS_PER_TILE)],
                    ztab_v.at[pl.ds(row0, ROWS_PER_TILE)])
    plsc.subcore_barrier()

    lanes = lax.broadcasted_iota(jnp.int32, (L,), 0)
    rots = [(lanes + r) % L for r in (8, 4, 2, 1)]

    def issue_idx(blk, s):
        pltpu.async_copy(sidx_hbm.at[wblk + blk], sidxb.at[s], isems[s])
        pltpu.async_copy(didx_hbm.at[wblk + blk], didxb.at[s], isems[s])

    def wait_idx(s):
        pltpu.make_async_copy(sidx_hbm.at[0], sidxb.at[s], isems[s]).wait()
        pltpu.make_async_copy(didx_hbm.at[0], didxb.at[s], isems[s]).wait()

    def issue_rows(isl, rs):
        pltpu.async_copy(ztab_v.at[sidxb.at[isl]], srows_v.at[rs], rsems[rs])
        pltpu.async_copy(ztab_v.at[didxb.at[isl]], drows_v.at[rs], rsems[rs])

    def wait_rows(isl, rs):
        pltpu.make_async_copy(
            ztab_v.at[sidxb.at[isl]], srows_v.at[rs], rsems[rs]).wait()
        pltpu.make_async_copy(
            ztab_v.at[didxb.at[isl]], drows_v.at[rs], rsems[rs]).wait()

    issue_idx(0, 0)
    issue_idx(1, 1)
    wait_idx(0)
    issue_rows(0, 0)

    def body(g, c):
        for s in range(NIS):
            blk = NIS * g + s
            rs = s % NRS
            sr = srows_v.at[rs]
            dr = drows_v.at[rs]
            ob = outb_v.at[rs]

            @pl.when(blk + 1 < NBLK)
            def _():
                wait_idx((s + 1) % NIS)
                issue_rows((s + 1) % NIS, (rs + 1) % NRS)

            @pl.when(blk + 2 < NBLK)
            def _():
                issue_idx(blk + 2, (s + 2) % NIS)

            wait_rows(s, rs)

            def grp_body(g2, c2):
                gbase = g2 * L

                def edge_body(i, res):
                    e = gbase + i
                    acc = jnp.zeros((L,), jnp.float32)
                    for j in range(DC):
                        sw = sr[e, pl.ds(j * L, L)]
                        dw = dr[e, pl.ds(j * L, L)]
                        sa = plsc.bitcast(sw << 16, jnp.float32)
                        sb = plsc.bitcast(sw & HIMASK, jnp.float32)
                        da = plsc.bitcast(dw << 16, jnp.float32)
                        db = plsc.bitcast(dw & HIMASK, jnp.float32)
                        acc = acc + sa * da + sb * db
                    for perm in rots:
                        acc = acc + _rotate(acc, perm)
                    return lax.select(lanes == i, acc, res)

                res = lax.fori_loop(0, L, edge_body,
                                    jnp.zeros((L,), jnp.float32), unroll=2)
                res = 1.0 / (1.0 + jnp.exp(-res))
                ob[pl.ds(pl.multiple_of(gbase, L), L)] = res
                return c2

            lax.fori_loop(0, B // L, grp_body, 0)

            pltpu.sync_copy(ob, out_hbm.at[wblk + blk])
        return c

    lax.fori_loop(0, NBLK // NIS, body, 0)


def _pack_bf16_words(z):
    zu = lax.bitcast_convert_type(z, jnp.uint32)
    lo = zu[:, :W]
    hi = zu[:, W:]

    def rne(x):
        return (x + jnp.uint32(0x7FFF) + ((x >> 16) & 1)) >> 16

    packed = rne(lo) | (rne(hi) << 16)
    words = lax.bitcast_convert_type(packed, jnp.int32)
    return jnp.pad(words, ((0, NP - N), (0, 0)))


def kernel(z, edge_idx):
    idx = edge_idx.astype(jnp.int32)
    pad = EP - E
    sidx = jnp.pad(idx[0], (0, pad)).reshape(EP // B, B)
    didx = jnp.pad(idx[1], (0, pad)).reshape(EP // B, B)
    out = _decode(_pack_bf16_words(z), sidx, didx)
    return out.reshape(-1)[:E]


# speedup vs baseline: 4.0880x; 1.1258x over previous
import functools

import jax
import jax.numpy as jnp
from jax import lax
from jax.experimental import pallas as pl
from jax.experimental.pallas import tpu as pltpu
from jax.experimental.pallas import tpu_sc as plsc

E = 160000
N = 10000
D = 256
W = D // 2
L = 16
NC, NS = 2, 16
NW = NC * NS
EP = 163840
PER_W = EP // NW
B = 64
NBLK = PER_W // B
NRS = 2
NIS = 4
DC = W // L
NP = 10240
ROWS_PER_TILE = NP // NS
HIMASK = -65536

_mesh = plsc.VectorSubcoreMesh(core_axis_name="c", subcore_axis_name="s")

_GATHER_DN = lax.GatherDimensionNumbers(
    offset_dims=(), collapsed_slice_dims=(0,), start_index_map=(0,))


def _rotate(v, perm):
    return lax.gather(v, perm[:, None], _GATHER_DN, slice_sizes=(1,),
                      mode=lax.GatherScatterMode.PROMISE_IN_BOUNDS)


@functools.partial(
    pl.kernel,
    mesh=_mesh,
    out_type=jax.ShapeDtypeStruct((EP // B, B), jnp.float32),
    compiler_params=pltpu.CompilerParams(needs_layout_passes=False),
    scratch_types=[
        pltpu.VMEM_SHARED((NP, W), jnp.int32),
        pltpu.VMEM((NIS, B), jnp.int32),
        pltpu.VMEM((NIS, B), jnp.int32),
        pltpu.VMEM((NRS, B, W), jnp.int32),
        pltpu.VMEM((NRS, B, W), jnp.int32),
        pltpu.VMEM((NRS, B), jnp.float32),
        pltpu.SemaphoreType.DMA,
        pltpu.SemaphoreType.DMA,
        pltpu.SemaphoreType.DMA,
        pltpu.SemaphoreType.DMA,
        pltpu.SemaphoreType.DMA,
        pltpu.SemaphoreType.DMA,
    ],
)
def _decode(z_hbm, sidx_hbm, didx_hbm, out_hbm,
            ztab_v, sidxb, didxb, srows_v, drows_v, outb_v, *sems):
    rsems = sems[:NRS]
    isems = sems[NRS:]
    cid = lax.axis_index("c")
    sid = lax.axis_index("s")
    wid = sid * NC + cid
    wblk = wid * NBLK

    row0 = sid * ROWS_PER_TILE
    pltpu.sync_copy(z_hbm.at[pl.ds(row0, ROWS_PER_TILE)],
                    ztab_v.at[pl.ds(row0, ROWS_PER_TILE)])
    plsc.subcore_barrier()

    lanes = lax.broadcasted_iota(jnp.int32, (L,), 0)
    rots = [(lanes + r) % L for r in (8, 4, 2, 1)]

    def issue_idx(blk, s):
        pltpu.async_copy(sidx_hbm.at[wblk + blk], sidxb.at[s], isems[s])
        pltpu.async_copy(didx_hbm.at[wblk + blk], didxb.at[s], isems[s])

    def wait_idx(s):
        pltpu.make_async_copy(sidx_hbm.at[0], sidxb.at[s], isems[s]).wait()
        pltpu.make_async_copy(didx_hbm.at[0], didxb.at[s], isems[s]).wait()

    def issue_rows(isl, rs):
        pltpu.async_copy(ztab_v.at[sidxb.at[isl]], srows_v.at[rs], rsems[rs])
        pltpu.async_copy(ztab_v.at[didxb.at[isl]], drows_v.at[rs], rsems[rs])

    def wait_rows(isl, rs):
        pltpu.make_async_copy(
            ztab_v.at[sidxb.at[isl]], srows_v.at[rs], rsems[rs]).wait()
        pltpu.make_async_copy(
            ztab_v.at[didxb.at[isl]], drows_v.at[rs], rsems[rs]).wait()

    issue_idx(0, 0)
    issue_idx(1, 1)
    wait_idx(0)
    issue_rows(0, 0)

    def body(g, c):
        for s in range(NIS):
            blk = NIS * g + s
            rs = s % NRS
            sr = srows_v.at[rs]
            dr = drows_v.at[rs]
            ob = outb_v.at[rs]

            @pl.when(blk + 1 < NBLK)
            def _():
                wait_idx((s + 1) % NIS)
                issue_rows((s + 1) % NIS, (rs + 1) % NRS)

            @pl.when(blk + 2 < NBLK)
            def _():
                issue_idx(blk + 2, (s + 2) % NIS)

            wait_rows(s, rs)

            def grp_body(g2, c2):
                gbase = g2 * L

                def edge_body(i, res):
                    e = gbase + i
                    acc = jnp.zeros((L,), jnp.float32)
                    for j in range(DC):
                        sw = sr[e, pl.ds(j * L, L)]
                        dw = dr[e, pl.ds(j * L, L)]
                        sa = plsc.bitcast(sw << 16, jnp.float32)
                        da = plsc.bitcast(dw << 16, jnp.float32)
                        sb = plsc.bitcast(sw, jnp.float32)
                        db = plsc.bitcast(dw, jnp.float32)
                        acc = acc + sa * da + sb * db
                    for perm in rots:
                        acc = acc + _rotate(acc, perm)
                    return lax.select(lanes == i, acc, res)

                res = lax.fori_loop(0, L, edge_body,
                                    jnp.zeros((L,), jnp.float32), unroll=2)
                res = 1.0 / (1.0 + jnp.exp(-res))
                ob[pl.ds(pl.multiple_of(gbase, L), L)] = res
                return c2

            lax.fori_loop(0, B // L, grp_body, 0)

            pltpu.sync_copy(ob, out_hbm.at[wblk + blk])
        return c

    lax.fori_loop(0, NBLK // NIS, body, 0)


def _pack_bf16_words(z):
    zu = lax.bitcast_convert_type(z, jnp.uint32)
    lo = zu[:, :W]
    hi = zu[:, W:]

    def rne(x):
        return (x + jnp.uint32(0x7FFF) + ((x >> 16) & 1)) >> 16

    packed = rne(lo) | (rne(hi) << 16)
    words = lax.bitcast_convert_type(packed, jnp.int32)
    return jnp.pad(words, ((0, NP - N), (0, 0)))


def kernel(z, edge_idx):
    idx = edge_idx.astype(jnp.int32)
    pad = EP - E
    sidx = jnp.pad(idx[0], (0, pad)).reshape(EP // B, B)
    didx = jnp.pad(idx[1], (0, pad)).reshape(EP // B, B)
    out = _decode(_pack_bf16_words(z), sidx, didx)
    return out.reshape(-1)[:E]
